# baseline (speedup 1.0000x reference)
import functools

import jax
import jax.numpy as jnp
from jax import lax
from jax.experimental import pallas as pl
from jax.experimental.pallas import tpu as pltpu
from jax.experimental.pallas import tpu_sc as plsc

NC = 2
NS = 16
NW = NC * NS
LANES = 16
CH = 80

_MESH = plsc.VectorSubcoreMesh(core_axis_name="c", subcore_axis_name="s")


def _make_norm_kernel(EROWS, ER, NR2):
    rps = EROWS // NS
    rpw = EROWS // NW
    zps = NR2 // NS
    KS = 20
    KG = 16

    @functools.partial(
        pl.kernel,
        out_type=jax.ShapeDtypeStruct((EROWS, 128), jnp.float32),
        mesh=_MESH,
        scratch_types=[
            pltpu.VMEM_SHARED((NR2,), jnp.float32),
            pltpu.VMEM((rps, 128), jnp.int32),
            pltpu.VMEM((128,), jnp.float32),
            pltpu.VMEM((KG, 128), jnp.float32),
            pltpu.VMEM((KG, 128), jnp.float32),
            pltpu.VMEM((zps,), jnp.float32),
            pltpu.SemaphoreType.DMA,
        ],
    )
    def norm_kernel(idx2d, norm_out, deg_sp, idxb, onesb, degb, normb,
                    zbuf, sem):
        c = lax.axis_index("c")
        s = lax.axis_index("s")
        wid = s * NC + c

        def fill_ones(i, carry):
            onesb[pl.ds(i * LANES, LANES)] = jnp.full((LANES,), 1.0,
                                                      jnp.float32)
            return carry

        lax.fori_loop(0, 128 // LANES, fill_ones, 0)

        def fill_zero(i, carry):
            zbuf[pl.ds(i * LANES, LANES)] = jnp.zeros((LANES,), jnp.float32)
            return carry

        lax.fori_loop(0, zps // LANES, fill_zero, 0)
        pltpu.sync_copy(zbuf, deg_sp.at[pl.ds(s * zps, zps)])
        plsc.subcore_barrier()

        pltpu.sync_copy(idx2d.at[pl.ds(s * rps, rps)], idxb)

        def dgroup(g, carry):
            j0 = g * KS
            descs = []
            for k in range(KS):
                descs.append(pltpu.async_copy(
                    onesb, deg_sp.at[idxb.at[j0 + k]], sem, add=True))
            for d in descs:
                d.wait()
            return carry

        lax.fori_loop(0, rps // KS, dgroup, 0)
        plsc.subcore_barrier()

        pltpu.sync_copy(idx2d.at[pl.ds(wid * rpw, rpw)],
                        idxb.at[pl.ds(0, rpw)])

        def ngroup(g, carry):
            j0 = g * KG
            descs = []
            for k in range(KG):
                descs.append(pltpu.async_copy(
                    deg_sp.at[idxb.at[j0 + k]], degb.at[k], sem))
            for d in descs:
                d.wait()
            gr0 = wid * rpw + j0
            for k in range(KG):
                mfac = jnp.where(gr0 + k < ER, 1.0, 0.0)
                for j in range(128 // LANES):
                    sl = pl.ds(j * LANES, LANES)
                    normb[k, sl] = (1.0 / degb[k, sl]) * mfac
            pltpu.sync_copy(normb, norm_out.at[pl.ds(gr0, KG)])
            return carry

        lax.fori_loop(0, rpw // KG, ngroup, 0)

    return norm_kernel


def _make_msg_kernel(N, D, EROWS):
    rpw = EROWS // NW
    BR = 16

    @functools.partial(
        pl.kernel,
        out_type=[jax.ShapeDtypeStruct((N, D), jnp.float32),
                  jax.ShapeDtypeStruct((N, D), jnp.float32)],
        mesh=_MESH,
        scratch_types=[
            pltpu.VMEM_SHARED((N, D), jnp.float32),
            pltpu.VMEM((BR, 128), jnp.int32),
            pltpu.VMEM((BR, 128), jnp.int32),
            pltpu.VMEM((BR, 128), jnp.float32),
            pltpu.VMEM((128, D), jnp.float32),
            pltpu.VMEM((128, D), jnp.float32),
            pltpu.VMEM((LANES, D), jnp.float32),
            pltpu.SemaphoreType.DMA,
            pltpu.SemaphoreType.DMA,
            pltpu.SemaphoreType.DMA,
            pltpu.SemaphoreType.DMA,
        ],
    )
    def msg_kernel(y, idx2d, dst2d, norm2d, out0, out1,
                   out_sp, idxb, dstb, normb, buf0, buf1, zbuf,
                   gs0, gs1, ss0, ss1):
        c = lax.axis_index("c")
        s = lax.axis_index("s")
        wid = s * NC + c

        def fill_zero(i, carry):
            for j in range(D // LANES):
                zbuf[i, pl.ds(j * LANES, LANES)] = jnp.zeros((LANES,),
                                                             jnp.float32)
            return carry

        nchunk = N // LANES
        with jax.named_scope("msg_zero"):
            lax.fori_loop(0, LANES, fill_zero, 0)

            def zchunk(k, carry):
                r0 = (s + k * NS) * LANES

                @pl.when(r0 < N)
                def _():
                    pltpu.sync_copy(zbuf, out_sp.at[pl.ds(r0, LANES)])

                return carry

            lax.fori_loop(0, (nchunk + NS - 1) // NS, zchunk, 0)
            plsc.subcore_barrier()

        bufs = (buf0, buf1)
        gsems = (gs0, gs1)
        ssems = (ss0, ss1)

        def blk(kb, carry):
            rbase = wid * rpw + kb * BR
            pltpu.sync_copy(idx2d.at[pl.ds(rbase, BR)], idxb)
            pltpu.sync_copy(dst2d.at[pl.ds(rbase, BR)], dstb)
            pltpu.sync_copy(norm2d.at[pl.ds(rbase, BR)], normb)

            gd = [None] * BR
            sd = [None] * BR
            gd[0] = pltpu.async_copy(y.at[idxb.at[0]], buf0, gs0)
            for i in range(BR):
                b = i % 2
                buf = bufs[b]
                if i + 1 < BR:
                    if i >= 1:
                        sd[i - 1].wait()
                    gd[i + 1] = pltpu.async_copy(
                        y.at[idxb.at[i + 1]], bufs[1 - b], gsems[1 - b])
                gd[i].wait()

                def scale(g2, inner, i=i, buf=buf):
                    e0 = g2 * LANES
                    nv16 = normb[i, pl.ds(e0, LANES)]
                    for t in range(LANES):
                        nv = nv16[t]
                        for j in range(D // LANES):
                            sl = pl.ds(j * LANES, LANES)
                            buf[e0 + t, sl] = buf[e0 + t, sl] * nv
                    return inner

                lax.fori_loop(0, 128 // LANES, scale, 0)
                sd[i] = pltpu.async_copy(buf, out_sp.at[dstb.at[i]],
                                         ssems[b], add=True)
            sd[BR - 2].wait()
            sd[BR - 1].wait()
            return carry

        with jax.named_scope("msg_edges"):
            lax.fori_loop(0, rpw // BR, blk, 0)
            plsc.subcore_barrier()

        with jax.named_scope("msg_out"):
            def out_chunk(k, carry):
                r0 = (s + k * NS) * LANES

                @pl.when(r0 < N)
                def _():
                    @pl.when(c == 0)
                    def _():
                        pltpu.sync_copy(out_sp.at[pl.ds(r0, LANES)],
                                        out0.at[pl.ds(r0, LANES)])

                    @pl.when(c == 1)
                    def _():
                        pltpu.sync_copy(out_sp.at[pl.ds(r0, LANES)],
                                        out1.at[pl.ds(r0, LANES)])

                return carry

            lax.fori_loop(0, (nchunk + NS - 1) // NS, out_chunk, 0)

    return msg_kernel


def _make_decode_kernel(N, D, Q):
    qpw = Q // NW

    @functools.partial(
        pl.kernel,
        out_type=jax.ShapeDtypeStruct((Q, LANES), jnp.float32),
        mesh=_MESH,
        scratch_types=[
            pltpu.VMEM((qpw,), jnp.int32),
            pltpu.VMEM((qpw,), jnp.int32),
            pltpu.VMEM((qpw,), jnp.int32),
            pltpu.VMEM((qpw, D), jnp.float32),
            pltpu.VMEM((qpw, D), jnp.float32),
            pltpu.VMEM((qpw, D), jnp.float32),
            pltpu.VMEM((qpw, LANES), jnp.float32),
            pltpu.SemaphoreType.DMA,
        ],
    )
    def decode_kernel(h, rel_emb, heads, rels, tails, scores,
                      hib, rib, tib, hrows, rrows, trows, outb, sem):
        c = lax.axis_index("c")
        s = lax.axis_index("s")
        wid = s * NC + c
        base = wid * qpw

        pltpu.sync_copy(heads.at[pl.ds(base, qpw)], hib)
        pltpu.sync_copy(rels.at[pl.ds(base, qpw)], rib)
        pltpu.sync_copy(tails.at[pl.ds(base, qpw)], tib)
        pltpu.async_copy(h.at[hib], hrows, sem).wait()
        pltpu.async_copy(rel_emb.at[rib], rrows, sem).wait()
        pltpu.async_copy(h.at[tib], trows, sem).wait()

        def one(q, carry):
            acc = jnp.zeros((LANES,), jnp.float32)
            for j in range(D // LANES):
                sl = pl.ds(j * LANES, LANES)
                acc = acc + (hrows[q, sl] * rrows[q, sl] * trows[q, sl])
            outb[q, :] = acc
            return carry

        lax.fori_loop(0, qpw, one, 0)
        pltpu.sync_copy(outb, scores.at[pl.ds(base, qpw)])

    return decode_kernel


def _wcat_body(att_ref, basis_ref, out_ref, *, R, NB, D):
    for r in range(R):
        acc = att_ref[r, 0] * basis_ref[0]
        for b in range(1, NB):
            acc = acc + att_ref[r, b] * basis_ref[b]
        out_ref[:, r * D:(r + 1) * D] = acc


def _wcat(att, basis):
    R, NB = att.shape
    D = basis.shape[-1]
    return pl.pallas_call(
        functools.partial(_wcat_body, R=R, NB=NB, D=D),
        out_shape=jax.ShapeDtypeStruct((D, R * D), jnp.float32),
        in_specs=[pl.BlockSpec(memory_space=pltpu.SMEM),
                  pl.BlockSpec((NB, D, D), lambda: (0, 0, 0))],
        out_specs=pl.BlockSpec((D, R * D), lambda: (0, 0)),
    )(att, basis)


def _mm_body(x_ref, w_ref, o_ref):
    o_ref[...] = jnp.dot(x_ref[...], w_ref[...],
                         preferred_element_type=jnp.float32,
                         precision=lax.Precision.HIGHEST)


def _mm(x, w, bn):
    n, d = x.shape
    m = w.shape[1]
    return pl.pallas_call(
        _mm_body,
        grid=(n // bn,),
        in_specs=[pl.BlockSpec((bn, d), lambda i: (i, 0)),
                  pl.BlockSpec((d, m), lambda i: (0, 0))],
        out_specs=pl.BlockSpec((bn, m), lambda i: (i, 0)),
        out_shape=jax.ShapeDtypeStruct((n, m), jnp.float32),
    )(x, w)


def _self_body(p0_ref, p1_ref, x_ref, w_ref, b_ref, o_ref, *, act):
    o = p0_ref[...] + p1_ref[...] + b_ref[...]
    o = o + jnp.dot(x_ref[...], w_ref[...],
                    preferred_element_type=jnp.float32,
                    precision=lax.Precision.HIGHEST)
    if act:
        o = jnp.maximum(o, 0.0)
    o_ref[...] = o


def _lane_sum_body(p_ref, o_ref):
    o_ref[...] = jnp.sum(p_ref[...], axis=-1)


def _lane_sum(prod):
    q, l = prod.shape
    return pl.pallas_call(
        _lane_sum_body,
        out_shape=jax.ShapeDtypeStruct((q,), jnp.float32),
    )(prod)


def _self_combine(p0, p1, x, w, bias, act, bn):
    n, d = x.shape
    return pl.pallas_call(
        functools.partial(_self_body, act=act),
        grid=(n // bn,),
        in_specs=[pl.BlockSpec((bn, d), lambda i: (i, 0)),
                  pl.BlockSpec((bn, d), lambda i: (i, 0)),
                  pl.BlockSpec((bn, d), lambda i: (i, 0)),
                  pl.BlockSpec((d, d), lambda i: (0, 0)),
                  pl.BlockSpec((1, d), lambda i: (0, 0))],
        out_specs=pl.BlockSpec((bn, d), lambda i: (i, 0)),
        out_shape=jax.ShapeDtypeStruct((n, d), jnp.float32),
    )(p0, p1, x, w, bias.reshape(1, d))


def kernel(edge_index, edge_type, heads, relations, tails, entity_emb,
           basis0, att0, self_w0, bias0, basis1, att1, self_w1, bias1,
           rel_emb):
    N, D = entity_emb.shape
    E = edge_type.shape[0]
    R = att0.shape[0]
    Q = heads.shape[0]
    NR = N * R
    BN = 400

    src = edge_index[0]
    dst = edge_index[1]
    idx_src = src * R + edge_type
    idx_deg = dst * R + edge_type

    rblock = NW * 16
    erows = ((E // 128 + rblock - 1) // rblock) * rblock
    e_pad = erows * 128
    er = E // 128
    pad = e_pad - E
    nr2 = ((NR // (NS * LANES)) + 1) * NS * LANES
    idx_src2d = jnp.concatenate(
        [idx_src, jnp.zeros((pad,), jnp.int32)]).reshape(erows, 128)
    dst2d = jnp.concatenate(
        [dst, jnp.zeros((pad,), jnp.int32)]).reshape(erows, 128)
    idx_deg2d = jnp.concatenate(
        [idx_deg, jnp.full((pad,), NR, jnp.int32)]).reshape(erows, 128)

    norm2d = _make_norm_kernel(erows, er, nr2)(idx_deg2d)
    msg = _make_msg_kernel(N, D, erows)

    x = entity_emb
    h = x
    for basis, att, self_w, bias, act in (
            (basis0, att0, self_w0, bias0, True),
            (basis1, att1, self_w1, bias1, False)):
        wcat = _wcat(att, basis)
        y = _mm(h, wcat, BN).reshape(N * R, D)
        p0, p1 = msg(y, idx_src2d, dst2d, norm2d)
        h = _self_combine(p0, p1, h, self_w, bias, act, BN)

    prod = _make_decode_kernel(N, D, Q)(h, rel_emb, heads, relations, tails)
    return _lane_sum(prod)


# speedup vs baseline: 1.0927x; 1.0927x over previous
import functools

import jax
import jax.numpy as jnp
from jax import lax
from jax.experimental import pallas as pl
from jax.experimental.pallas import tpu as pltpu
from jax.experimental.pallas import tpu_sc as plsc

NC = 2
NS = 16
NW = NC * NS
LANES = 16
CH = 80

_MESH = plsc.VectorSubcoreMesh(core_axis_name="c", subcore_axis_name="s")


def _make_norm_kernel(EROWS, ER, NR2):
    rps = EROWS // NS
    rpw = EROWS // NW
    zps = NR2 // NS
    KS = 20
    KG = 16

    @functools.partial(
        pl.kernel,
        out_type=jax.ShapeDtypeStruct((EROWS, 128), jnp.float32),
        mesh=_MESH,
        scratch_types=[
            pltpu.VMEM_SHARED((NR2,), jnp.float32),
            pltpu.VMEM((rps, 128), jnp.int32),
            pltpu.VMEM((128,), jnp.float32),
            pltpu.VMEM((KG, 128), jnp.float32),
            pltpu.VMEM((KG, 128), jnp.float32),
            pltpu.VMEM((zps,), jnp.float32),
            pltpu.SemaphoreType.DMA,
        ],
    )
    def norm_kernel(idx2d, norm_out, deg_sp, idxb, onesb, degb, normb,
                    zbuf, sem):
        c = lax.axis_index("c")
        s = lax.axis_index("s")
        wid = s * NC + c

        def fill_ones(i, carry):
            onesb[pl.ds(i * LANES, LANES)] = jnp.full((LANES,), 1.0,
                                                      jnp.float32)
            return carry

        lax.fori_loop(0, 128 // LANES, fill_ones, 0)

        def fill_zero(i, carry):
            zbuf[pl.ds(i * LANES, LANES)] = jnp.zeros((LANES,), jnp.float32)
            return carry

        lax.fori_loop(0, zps // LANES, fill_zero, 0)
        pltpu.sync_copy(zbuf, deg_sp.at[pl.ds(s * zps, zps)])
        plsc.subcore_barrier()

        pltpu.sync_copy(idx2d.at[pl.ds(s * rps, rps)], idxb)

        def dgroup(g, carry):
            j0 = g * KS
            descs = []
            for k in range(KS):
                descs.append(pltpu.async_copy(
                    onesb, deg_sp.at[idxb.at[j0 + k]], sem, add=True))
            for d in descs:
                d.wait()
            return carry

        lax.fori_loop(0, rps // KS, dgroup, 0)
        plsc.subcore_barrier()

        pltpu.sync_copy(idx2d.at[pl.ds(wid * rpw, rpw)],
                        idxb.at[pl.ds(0, rpw)])

        def ngroup(g, carry):
            j0 = g * KG
            descs = []
            for k in range(KG):
                descs.append(pltpu.async_copy(
                    deg_sp.at[idxb.at[j0 + k]], degb.at[k], sem))
            for d in descs:
                d.wait()
            gr0 = wid * rpw + j0
            for k in range(KG):
                mfac = jnp.where(gr0 + k < ER, 1.0, 0.0)
                for j in range(128 // LANES):
                    sl = pl.ds(j * LANES, LANES)
                    normb[k, sl] = (1.0 / degb[k, sl]) * mfac
            pltpu.sync_copy(normb, norm_out.at[pl.ds(gr0, KG)])
            return carry

        lax.fori_loop(0, rpw // KG, ngroup, 0)

    return norm_kernel


FAST_CORE = 0
FAST_FRAC_NUM, FAST_FRAC_DEN = 4, 5


def _make_msg_kernel(N, D, EROWS):
    rpw = EROWS // NW
    BR = 16
    rcf = (EROWS * FAST_FRAC_NUM // FAST_FRAC_DEN) // (NS * BR) * (NS * BR)
    rcs = EROWS - rcf
    assert rcs % (NS * BR) == 0

    @functools.partial(
        pl.kernel,
        out_type=[jax.ShapeDtypeStruct((N, D), jnp.float32),
                  jax.ShapeDtypeStruct((N, D), jnp.float32)],
        mesh=_MESH,
        scratch_types=[
            pltpu.VMEM_SHARED((N, D), jnp.float32),
            pltpu.VMEM((BR, 128), jnp.int32),
            pltpu.VMEM((BR, 128), jnp.int32),
            pltpu.VMEM((BR, 128), jnp.float32),
            pltpu.VMEM((128, D), jnp.float32),
            pltpu.VMEM((128, D), jnp.float32),
            pltpu.VMEM((LANES, D), jnp.float32),
            pltpu.SemaphoreType.DMA,
            pltpu.SemaphoreType.DMA,
            pltpu.SemaphoreType.DMA,
            pltpu.SemaphoreType.DMA,
        ],
    )
    def msg_kernel(y, idx2d, dst2d, norm2d, out0, out1,
                   out_sp, idxb, dstb, normb, buf0, buf1, zbuf,
                   gs0, gs1, ss0, ss1):
        c = lax.axis_index("c")
        s = lax.axis_index("s")
        wid = s * NC + c

        def fill_zero(i, carry):
            for j in range(D // LANES):
                zbuf[i, pl.ds(j * LANES, LANES)] = jnp.zeros((LANES,),
                                                             jnp.float32)
            return carry

        nchunk = N // LANES
        with jax.named_scope("msg_zero"):
            lax.fori_loop(0, LANES, fill_zero, 0)

            def zchunk(k, carry):
                r0 = (s + k * NS) * LANES

                @pl.when(r0 < N)
                def _():
                    pltpu.sync_copy(zbuf, out_sp.at[pl.ds(r0, LANES)])

                return carry

            lax.fori_loop(0, (nchunk + NS - 1) // NS, zchunk, 0)
            plsc.subcore_barrier()

        bufs = (buf0, buf1)
        gsems = (gs0, gs1)
        ssems = (ss0, ss1)

        def blk_at(rbase):
            pltpu.sync_copy(idx2d.at[pl.ds(rbase, BR)], idxb)
            pltpu.sync_copy(dst2d.at[pl.ds(rbase, BR)], dstb)
            pltpu.sync_copy(norm2d.at[pl.ds(rbase, BR)], normb)

            gd = [None] * BR
            sd = [None] * BR
            gd[0] = pltpu.async_copy(y.at[idxb.at[0]], buf0, gs0)
            for i in range(BR):
                b = i % 2
                buf = bufs[b]
                if i + 1 < BR:
                    if i >= 1:
                        sd[i - 1].wait()
                    gd[i + 1] = pltpu.async_copy(
                        y.at[idxb.at[i + 1]], bufs[1 - b], gsems[1 - b])
                gd[i].wait()

                def scale(g2, inner, i=i, buf=buf):
                    e0 = g2 * LANES
                    nv16 = normb[i, pl.ds(e0, LANES)]
                    for t in range(LANES):
                        nv = nv16[t]
                        for j in range(D // LANES):
                            sl = pl.ds(j * LANES, LANES)
                            buf[e0 + t, sl] = buf[e0 + t, sl] * nv
                    return inner

                lax.fori_loop(0, 128 // LANES, scale, 0)
                sd[i] = pltpu.async_copy(buf, out_sp.at[dstb.at[i]],
                                         ssems[b], add=True)
            sd[BR - 2].wait()
            sd[BR - 1].wait()

        with jax.named_scope("msg_edges"):
            fast = c == FAST_CORE
            base0 = jnp.where(fast, s * (rcf // NS), rcf + s * (rcs // NS))
            nblk = jnp.where(fast, rcf // (NS * BR), rcs // (NS * BR))

            def blkl(kb, carry):
                blk_at(base0 + kb * BR)
                return carry

            lax.fori_loop(0, nblk, blkl, 0)
            plsc.subcore_barrier()

        with jax.named_scope("msg_out"):
            def out_chunk(k, carry):
                r0 = (s + k * NS) * LANES

                @pl.when(r0 < N)
                def _():
                    @pl.when(c == 0)
                    def _():
                        pltpu.sync_copy(out_sp.at[pl.ds(r0, LANES)],
                                        out0.at[pl.ds(r0, LANES)])

                    @pl.when(c == 1)
                    def _():
                        pltpu.sync_copy(out_sp.at[pl.ds(r0, LANES)],
                                        out1.at[pl.ds(r0, LANES)])

                return carry

            lax.fori_loop(0, (nchunk + NS - 1) // NS, out_chunk, 0)

    return msg_kernel


def _make_decode_kernel(N, D, Q):
    qpw = Q // NW

    @functools.partial(
        pl.kernel,
        out_type=jax.ShapeDtypeStruct((Q, LANES), jnp.float32),
        mesh=_MESH,
        scratch_types=[
            pltpu.VMEM((qpw,), jnp.int32),
            pltpu.VMEM((qpw,), jnp.int32),
            pltpu.VMEM((qpw,), jnp.int32),
            pltpu.VMEM((qpw, D), jnp.float32),
            pltpu.VMEM((qpw, D), jnp.float32),
            pltpu.VMEM((qpw, D), jnp.float32),
            pltpu.VMEM((qpw, LANES), jnp.float32),
            pltpu.SemaphoreType.DMA,
        ],
    )
    def decode_kernel(h, rel_emb, heads, rels, tails, scores,
                      hib, rib, tib, hrows, rrows, trows, outb, sem):
        c = lax.axis_index("c")
        s = lax.axis_index("s")
        wid = s * NC + c
        base = wid * qpw

        pltpu.sync_copy(heads.at[pl.ds(base, qpw)], hib)
        pltpu.sync_copy(rels.at[pl.ds(base, qpw)], rib)
        pltpu.sync_copy(tails.at[pl.ds(base, qpw)], tib)
        pltpu.async_copy(h.at[hib], hrows, sem).wait()
        pltpu.async_copy(rel_emb.at[rib], rrows, sem).wait()
        pltpu.async_copy(h.at[tib], trows, sem).wait()

        def one(q, carry):
            acc = jnp.zeros((LANES,), jnp.float32)
            for j in range(D // LANES):
                sl = pl.ds(j * LANES, LANES)
                acc = acc + (hrows[q, sl] * rrows[q, sl] * trows[q, sl])
            outb[q, :] = acc
            return carry

        lax.fori_loop(0, qpw, one, 0)
        pltpu.sync_copy(outb, scores.at[pl.ds(base, qpw)])

    return decode_kernel


def _wcat_body(att_ref, basis_ref, out_ref, *, R, NB, D):
    for r in range(R):
        acc = att_ref[r, 0] * basis_ref[0]
        for b in range(1, NB):
            acc = acc + att_ref[r, b] * basis_ref[b]
        out_ref[:, r * D:(r + 1) * D] = acc


def _wcat(att, basis):
    R, NB = att.shape
    D = basis.shape[-1]
    return pl.pallas_call(
        functools.partial(_wcat_body, R=R, NB=NB, D=D),
        out_shape=jax.ShapeDtypeStruct((D, R * D), jnp.float32),
        in_specs=[pl.BlockSpec(memory_space=pltpu.SMEM),
                  pl.BlockSpec((NB, D, D), lambda: (0, 0, 0))],
        out_specs=pl.BlockSpec((D, R * D), lambda: (0, 0)),
    )(att, basis)


def _mm_body(x_ref, w_ref, o_ref):
    o_ref[...] = jnp.dot(x_ref[...], w_ref[...],
                         preferred_element_type=jnp.float32,
                         precision=lax.Precision.HIGHEST)


def _mm(x, w, bn):
    n, d = x.shape
    m = w.shape[1]
    return pl.pallas_call(
        _mm_body,
        grid=(n // bn,),
        in_specs=[pl.BlockSpec((bn, d), lambda i: (i, 0)),
                  pl.BlockSpec((d, m), lambda i: (0, 0))],
        out_specs=pl.BlockSpec((bn, m), lambda i: (i, 0)),
        out_shape=jax.ShapeDtypeStruct((n, m), jnp.float32),
    )(x, w)


def _self_body(p0_ref, p1_ref, x_ref, w_ref, b_ref, o_ref, *, act):
    o = p0_ref[...] + p1_ref[...] + b_ref[...]
    o = o + jnp.dot(x_ref[...], w_ref[...],
                    preferred_element_type=jnp.float32,
                    precision=lax.Precision.HIGHEST)
    if act:
        o = jnp.maximum(o, 0.0)
    o_ref[...] = o


def _lane_sum_body(p_ref, o_ref):
    o_ref[...] = jnp.sum(p_ref[...], axis=-1)


def _lane_sum(prod):
    q, l = prod.shape
    return pl.pallas_call(
        _lane_sum_body,
        out_shape=jax.ShapeDtypeStruct((q,), jnp.float32),
    )(prod)


def _self_combine(p0, p1, x, w, bias, act, bn):
    n, d = x.shape
    return pl.pallas_call(
        functools.partial(_self_body, act=act),
        grid=(n // bn,),
        in_specs=[pl.BlockSpec((bn, d), lambda i: (i, 0)),
                  pl.BlockSpec((bn, d), lambda i: (i, 0)),
                  pl.BlockSpec((bn, d), lambda i: (i, 0)),
                  pl.BlockSpec((d, d), lambda i: (0, 0)),
                  pl.BlockSpec((1, d), lambda i: (0, 0))],
        out_specs=pl.BlockSpec((bn, d), lambda i: (i, 0)),
        out_shape=jax.ShapeDtypeStruct((n, d), jnp.float32),
    )(p0, p1, x, w, bias.reshape(1, d))


def kernel(edge_index, edge_type, heads, relations, tails, entity_emb,
           basis0, att0, self_w0, bias0, basis1, att1, self_w1, bias1,
           rel_emb):
    N, D = entity_emb.shape
    E = edge_type.shape[0]
    R = att0.shape[0]
    Q = heads.shape[0]
    NR = N * R
    BN = 400

    src = edge_index[0]
    dst = edge_index[1]
    idx_src = src * R + edge_type
    idx_deg = dst * R + edge_type

    rblock = NW * 16
    erows = ((E // 128 + rblock - 1) // rblock) * rblock
    e_pad = erows * 128
    er = E // 128
    pad = e_pad - E
    nr2 = ((NR // (NS * LANES)) + 1) * NS * LANES
    idx_src2d = jnp.concatenate(
        [idx_src, jnp.zeros((pad,), jnp.int32)]).reshape(erows, 128)
    dst2d = jnp.concatenate(
        [dst, jnp.zeros((pad,), jnp.int32)]).reshape(erows, 128)
    idx_deg2d = jnp.concatenate(
        [idx_deg, jnp.full((pad,), NR, jnp.int32)]).reshape(erows, 128)

    norm2d = _make_norm_kernel(erows, er, nr2)(idx_deg2d)
    msg = _make_msg_kernel(N, D, erows)

    x = entity_emb
    h = x
    for basis, att, self_w, bias, act in (
            (basis0, att0, self_w0, bias0, True),
            (basis1, att1, self_w1, bias1, False)):
        wcat = _wcat(att, basis)
        y = _mm(h, wcat, BN).reshape(N * R, D)
        p0, p1 = msg(y, idx_src2d, dst2d, norm2d)
        h = _self_combine(p0, p1, h, self_w, bias, act, BN)

    prod = _make_decode_kernel(N, D, Q)(h, rel_emb, heads, relations, tails)
    return _lane_sum(prod)
